# baseline (speedup 1.0000x reference)
import jax
import jax.numpy as jnp
from jax import lax
from jax.experimental import pallas as pl
from jax.experimental.pallas import tpu as pltpu
from jax.experimental.pallas import tpu_sc as plsc

B = 4096
H = 50
D = 64
NCLS = 1000
V = 1_000_000
NW = 32
BPW = B // NW

CH = 256
PR = CH // 2
VFULL = 999936
NCH = VFULL // CH
KPW = NCH // NW


NT = CH // 128


def _transpose_chunk(bin_ref, bout_ref, pat0):
    def per_s(s, carry):
        pat_s = pat0 + s
        for r in range(8):
            for t in range(NT):
                for g in range(8):
                    x = bin_ref[r, s, pl.ds(t * 128 + g * 16, 16)]
                    plsc.store_scatter(
                        bout_ref.at[pl.ds((t * 128 + g * 16) * D, 16 * D)],
                        [pat_s + (8 * r)], x)
        return carry

    lax.fori_loop(0, 8, per_s, 0)


def _stage_in(table_t, bin_ref, c, sem):
    cps = []
    for r in range(8):
        cps.append(pltpu.async_copy(
            table_t.at[pl.ds(8 * r, 8), pl.ds(c * CH, CH)],
            bin_ref.at[r], sem))
    return cps


def _sc_transpose_body(table_t, tail2, out, bin0, bin1, bout0, bout1,
                       tailb, sin0, sin1, sout0, sout1, stail):
    wid = lax.axis_index("s") * 2 + lax.axis_index("c")
    iota = lax.iota(jnp.int32, 16)
    pat0 = iota * D

    @pl.when(wid == 0)
    def _tail():
        cp = pltpu.async_copy(tail2, tailb, stail)
        cp.wait()
        pltpu.sync_copy(tailb, out.at[pl.ds(VFULL * D, 64 * D)])

    def chunk_pair(k, carry):
        c0 = wid + (2 * k) * NW
        c1 = wid + (2 * k + 1) * NW
        cp0 = _stage_in(table_t, bin0, c0, sin0)
        cp1 = _stage_in(table_t, bin1, c1, sin1)
        for cp in cp0:
            cp.wait()
        _transpose_chunk(bin0, bout0, pat0)
        o0 = pltpu.async_copy(bout0, out.at[pl.ds(c0 * CH * D, CH * D)], sout0)
        for cp in cp1:
            cp.wait()
        _transpose_chunk(bin1, bout1, pat0)
        o1 = pltpu.async_copy(bout1, out.at[pl.ds(c1 * CH * D, CH * D)], sout1)
        o0.wait()
        o1.wait()
        return carry

    lax.fori_loop(0, KPW // 2, chunk_pair, 0)

    @pl.when(wid < 2)
    def _leftover():
        c = NCH - 2 + wid
        cps = _stage_in(table_t, bin0, c, sin0)
        for cp in cps:
            cp.wait()
        _transpose_chunk(bin0, bout0, pat0)
        pltpu.sync_copy(bout0, out.at[pl.ds(c * CH * D, CH * D)])


def _sc_transpose(table_t, tail2):
    return pl.kernel(
        _sc_transpose_body,
        out_type=jax.ShapeDtypeStruct((V * D,), jnp.float32),
        mesh=plsc.VectorSubcoreMesh(core_axis_name="c", subcore_axis_name="s"),
        scratch_types=[
            pltpu.VMEM((8, 8, CH), jnp.float32),
            pltpu.VMEM((8, 8, CH), jnp.float32),
            pltpu.VMEM((CH * D,), jnp.float32),
            pltpu.VMEM((CH * D,), jnp.float32),
            pltpu.VMEM((64 * D,), jnp.float32),
            pltpu.SemaphoreType.DMA,
            pltpu.SemaphoreType.DMA,
            pltpu.SemaphoreType.DMA,
            pltpu.SemaphoreType.DMA,
            pltpu.SemaphoreType.DMA,
        ],
        compiler_params=pltpu.CompilerParams(needs_layout_passes=False),
    )(table_t, tail2)


def _sc_pool_body(ids_t, table, out, idsv, acc_a, acc_b, sem_a, sem_b):
    wid = lax.axis_index("s") * 2 + lax.axis_index("c")
    base = wid * BPW
    pltpu.sync_copy(ids_t.at[:, pl.ds(base, BPW)], idsv)
    cp_a = pltpu.async_copy(table.at[idsv.at[0]], acc_a, sem_a)
    cp_b = pltpu.async_copy(table.at[idsv.at[1]], acc_b, sem_b)
    for j in range(2, H, 2):
        cp_a.wait()
        cp_a = pltpu.async_copy(table.at[idsv.at[j]], acc_a, sem_a, add=True)
        if j + 1 < H:
            cp_b.wait()
            cp_b = pltpu.async_copy(table.at[idsv.at[j + 1]], acc_b, sem_b,
                                    add=True)
    cp_a.wait()
    cp_b.wait()

    def merge(i, carry):
        r = i // (D // 16)
        c = (i % (D // 16)) * 16
        acc_a[r, pl.ds(c, 16)] = acc_a[r, pl.ds(c, 16)] + acc_b[r, pl.ds(c, 16)]
        return carry

    lax.fori_loop(0, BPW * (D // 16), merge, 0)
    pltpu.sync_copy(acc_a, out.at[pl.ds(base, BPW), :])


def _sc_pool(ids_t, table):
    return pl.kernel(
        _sc_pool_body,
        out_type=jax.ShapeDtypeStruct((B, D), jnp.float32),
        mesh=plsc.VectorSubcoreMesh(core_axis_name="c", subcore_axis_name="s"),
        scratch_types=[
            pltpu.VMEM((H, BPW), jnp.int32),
            pltpu.VMEM((BPW, D), jnp.float32),
            pltpu.VMEM((BPW, D), jnp.float32),
            pltpu.SemaphoreType.DMA,
            pltpu.SemaphoreType.DMA,
        ],
        compiler_params=pltpu.CompilerParams(use_tc_tiling_on_sc=False),
    )(ids_t, table)


def _tc_body(acc_ref, ids_ref, w_ref, b_ref, t0_ref, out_ref):
    acc = acc_ref[...]
    ids = ids_ref[...]
    z = jnp.sum((ids == 0).astype(jnp.float32), axis=1, keepdims=True)
    w = w_ref[...]
    t0 = t0_ref[...]
    w0 = lax.dot_general(t0, w, (((1,), (1,)), ((), ())),
                         precision=lax.Precision.HIGHEST,
                         preferred_element_type=jnp.float32)
    y = lax.dot_general(acc, w, (((1,), (1,)), ((), ())),
                        precision=lax.Precision.HIGHEST,
                        preferred_element_type=jnp.float32)
    out_ref[...] = y + b_ref[...] - z * w0


_TC_BLK = 512


def _tc_head(acc, ids, w_out, b_out2, t0):
    return pl.pallas_call(
        _tc_body,
        grid=(B // _TC_BLK,),
        in_specs=[
            pl.BlockSpec((_TC_BLK, D), lambda i: (i, 0)),
            pl.BlockSpec((_TC_BLK, H), lambda i: (i, 0)),
            pl.BlockSpec((NCLS, D), lambda i: (0, 0)),
            pl.BlockSpec((1, NCLS), lambda i: (0, 0)),
            pl.BlockSpec((1, D), lambda i: (0, 0)),
        ],
        out_specs=pl.BlockSpec((_TC_BLK, NCLS), lambda i: (i, 0)),
        out_shape=jax.ShapeDtypeStruct((B, NCLS), jnp.float32),
    )(acc, ids, w_out, b_out2, t0)


def kernel(words_as_ids, table, W_out, b_out):
    ids = words_as_ids.astype(jnp.int32)
    ids_t = ids.T
    table_t = table.T
    tail2 = lax.slice(table, (VFULL, 0), (V, D)).reshape(64 * D)
    table_l = _sc_transpose(table_t, tail2)
    table_r = jnp.reshape(table_l, (V, D))
    acc = _sc_pool(ids_t, table_r)
    t0 = lax.slice(table_l, (0,), (D,)).reshape(1, D)
    b2 = b_out.reshape(1, NCLS)
    return _tc_head(acc, ids, W_out, b2, t0)


# speedup vs baseline: 1.0625x; 1.0625x over previous
import jax
import jax.numpy as jnp
from jax import lax
from jax.experimental import pallas as pl
from jax.experimental.pallas import tpu as pltpu
from jax.experimental.pallas import tpu_sc as plsc

B = 4096
H = 50
D = 64
NCLS = 1000
V = 1_000_000
NW = 32
BPW = B // NW

CH = 256
PR = CH // 2
VFULL = 999936
NCH = VFULL // CH
KPW = NCH // NW


NT = CH // 128


def _transpose_chunk(bin_ref, bout_ref, pat0):
    items = [(r, t, g) for r in range(8) for t in range(NT) for g in range(8)]
    PIPE = 8

    def per_s(s, carry):
        pat_s = pat0 + s
        q = []

        def flush_one():
            (r, t, g), x = q.pop(0)
            plsc.store_scatter(
                bout_ref.at[pl.ds((t * 128 + g * 16) * D, 16 * D)],
                [pat_s + (8 * r)], x)

        for it in items:
            r, t, g = it
            q.append((it, bin_ref[r, s, pl.ds(t * 128 + g * 16, 16)]))
            if len(q) >= PIPE:
                flush_one()
        while q:
            flush_one()
        return carry

    lax.fori_loop(0, 8, per_s, 0)


def _stage_in(table_t, bin_ref, c, sem):
    cps = []
    for r in range(8):
        cps.append(pltpu.async_copy(
            table_t.at[pl.ds(8 * r, 8), pl.ds(c * CH, CH)],
            bin_ref.at[r], sem))
    return cps


def _sc_transpose_body(table_t, tail2, out, bin0, bin1, bout0, bout1,
                       tailb, sin0, sin1, sout0, sout1, stail):
    wid = lax.axis_index("s") * 2 + lax.axis_index("c")
    iota = lax.iota(jnp.int32, 16)
    pat0 = iota * D

    @pl.when(wid == 0)
    def _tail():
        cp = pltpu.async_copy(tail2, tailb, stail)
        cp.wait()
        pltpu.sync_copy(tailb, out.at[pl.ds(VFULL * D, 64 * D)])

    def chunk_pair(k, carry):
        c0 = wid + (2 * k) * NW
        c1 = wid + (2 * k + 1) * NW
        cp0 = _stage_in(table_t, bin0, c0, sin0)
        cp1 = _stage_in(table_t, bin1, c1, sin1)
        for cp in cp0:
            cp.wait()
        _transpose_chunk(bin0, bout0, pat0)
        o0 = pltpu.async_copy(bout0, out.at[pl.ds(c0 * CH * D, CH * D)], sout0)
        for cp in cp1:
            cp.wait()
        _transpose_chunk(bin1, bout1, pat0)
        o1 = pltpu.async_copy(bout1, out.at[pl.ds(c1 * CH * D, CH * D)], sout1)
        o0.wait()
        o1.wait()
        return carry

    lax.fori_loop(0, KPW // 2, chunk_pair, 0)

    @pl.when(wid < 2)
    def _leftover():
        c = NCH - 2 + wid
        cps = _stage_in(table_t, bin0, c, sin0)
        for cp in cps:
            cp.wait()
        _transpose_chunk(bin0, bout0, pat0)
        pltpu.sync_copy(bout0, out.at[pl.ds(c * CH * D, CH * D)])


def _sc_transpose(table_t, tail2):
    return pl.kernel(
        _sc_transpose_body,
        out_type=jax.ShapeDtypeStruct((V * D,), jnp.float32),
        mesh=plsc.VectorSubcoreMesh(core_axis_name="c", subcore_axis_name="s"),
        scratch_types=[
            pltpu.VMEM((8, 8, CH), jnp.float32),
            pltpu.VMEM((8, 8, CH), jnp.float32),
            pltpu.VMEM((CH * D,), jnp.float32),
            pltpu.VMEM((CH * D,), jnp.float32),
            pltpu.VMEM((64 * D,), jnp.float32),
            pltpu.SemaphoreType.DMA,
            pltpu.SemaphoreType.DMA,
            pltpu.SemaphoreType.DMA,
            pltpu.SemaphoreType.DMA,
            pltpu.SemaphoreType.DMA,
        ],
        compiler_params=pltpu.CompilerParams(needs_layout_passes=False),
    )(table_t, tail2)


def _sc_pool_body(ids_t, table, out, idsv, acc_a, acc_b, sem_a, sem_b):
    wid = lax.axis_index("s") * 2 + lax.axis_index("c")
    base = wid * BPW
    pltpu.sync_copy(ids_t.at[:, pl.ds(base, BPW)], idsv)
    cp_a = pltpu.async_copy(table.at[idsv.at[0]], acc_a, sem_a)
    cp_b = pltpu.async_copy(table.at[idsv.at[1]], acc_b, sem_b)
    for j in range(2, H, 2):
        cp_a.wait()
        cp_a = pltpu.async_copy(table.at[idsv.at[j]], acc_a, sem_a, add=True)
        if j + 1 < H:
            cp_b.wait()
            cp_b = pltpu.async_copy(table.at[idsv.at[j + 1]], acc_b, sem_b,
                                    add=True)
    cp_a.wait()
    cp_b.wait()

    def merge(i, carry):
        r = i // (D // 16)
        c = (i % (D // 16)) * 16
        acc_a[r, pl.ds(c, 16)] = acc_a[r, pl.ds(c, 16)] + acc_b[r, pl.ds(c, 16)]
        return carry

    lax.fori_loop(0, BPW * (D // 16), merge, 0)
    pltpu.sync_copy(acc_a, out.at[pl.ds(base, BPW), :])


def _sc_pool(ids_t, table):
    return pl.kernel(
        _sc_pool_body,
        out_type=jax.ShapeDtypeStruct((B, D), jnp.float32),
        mesh=plsc.VectorSubcoreMesh(core_axis_name="c", subcore_axis_name="s"),
        scratch_types=[
            pltpu.VMEM((H, BPW), jnp.int32),
            pltpu.VMEM((BPW, D), jnp.float32),
            pltpu.VMEM((BPW, D), jnp.float32),
            pltpu.SemaphoreType.DMA,
            pltpu.SemaphoreType.DMA,
        ],
        compiler_params=pltpu.CompilerParams(use_tc_tiling_on_sc=False),
    )(ids_t, table)


def _tc_body(acc_ref, ids_ref, w_ref, b_ref, t0_ref, out_ref):
    acc = acc_ref[...]
    ids = ids_ref[...]
    z = jnp.sum((ids == 0).astype(jnp.float32), axis=1, keepdims=True)
    w = w_ref[...]
    t0 = t0_ref[...]
    w0 = lax.dot_general(t0, w, (((1,), (1,)), ((), ())),
                         precision=lax.Precision.HIGHEST,
                         preferred_element_type=jnp.float32)
    y = lax.dot_general(acc, w, (((1,), (1,)), ((), ())),
                        precision=lax.Precision.HIGHEST,
                        preferred_element_type=jnp.float32)
    out_ref[...] = y + b_ref[...] - z * w0


_TC_BLK = 512


def _tc_head(acc, ids, w_out, b_out2, t0):
    return pl.pallas_call(
        _tc_body,
        grid=(B // _TC_BLK,),
        in_specs=[
            pl.BlockSpec((_TC_BLK, D), lambda i: (i, 0)),
            pl.BlockSpec((_TC_BLK, H), lambda i: (i, 0)),
            pl.BlockSpec((NCLS, D), lambda i: (0, 0)),
            pl.BlockSpec((1, NCLS), lambda i: (0, 0)),
            pl.BlockSpec((1, D), lambda i: (0, 0)),
        ],
        out_specs=pl.BlockSpec((_TC_BLK, NCLS), lambda i: (i, 0)),
        out_shape=jax.ShapeDtypeStruct((B, NCLS), jnp.float32),
    )(acc, ids, w_out, b_out2, t0)


def kernel(words_as_ids, table, W_out, b_out):
    ids = words_as_ids.astype(jnp.int32)
    ids_t = ids.T
    table_t = table.T
    tail2 = lax.slice(table, (VFULL, 0), (V, D)).reshape(64 * D)
    table_l = _sc_transpose(table_t, tail2)
    table_r = jnp.reshape(table_l, (V, D))
    acc = _sc_pool(ids_t, table_r)
    t0 = lax.slice(table_l, (0,), (D,)).reshape(1, D)
    b2 = b_out.reshape(1, NCLS)
    return _tc_head(acc, ids, W_out, b2, t0)


# speedup vs baseline: 2.2908x; 2.1562x over previous
import jax
import jax.numpy as jnp
from jax import lax
from jax.experimental import pallas as pl
from jax.experimental.pallas import tpu as pltpu
from jax.experimental.pallas import tpu_sc as plsc

B = 4096
H = 50
D = 64
DP = 128
NCLS = 1000
NW = 32
BPW = B // NW


def _sc_pool_body(ids_t, table, out, idsv, acc_a, acc_b, sem_a, sem_b):
    wid = lax.axis_index("s") * 2 + lax.axis_index("c")
    base = wid * BPW
    pltpu.sync_copy(ids_t.at[:, pl.ds(base, BPW)], idsv)
    cp_a = pltpu.async_copy(table.at[idsv.at[0]], acc_a, sem_a)
    cp_b = pltpu.async_copy(table.at[idsv.at[1]], acc_b, sem_b)
    for j in range(2, H, 2):
        cp_a.wait()
        cp_a = pltpu.async_copy(table.at[idsv.at[j]], acc_a, sem_a, add=True)
        if j + 1 < H:
            cp_b.wait()
            cp_b = pltpu.async_copy(table.at[idsv.at[j + 1]], acc_b, sem_b,
                                    add=True)
    cp_a.wait()
    cp_b.wait()

    def merge(i, carry):
        r = i // (D // 16)
        c = (i % (D // 16)) * 16
        acc_a[r, pl.ds(c, 16)] = acc_a[r, pl.ds(c, 16)] + acc_b[r, pl.ds(c, 16)]
        return carry

    lax.fori_loop(0, BPW * (D // 16), merge, 0)
    pltpu.sync_copy(acc_a, out.at[pl.ds(base, BPW), :])


def _sc_pool(ids_t, table128):
    return pl.kernel(
        _sc_pool_body,
        out_type=jax.ShapeDtypeStruct((B, DP), jnp.float32),
        mesh=plsc.VectorSubcoreMesh(core_axis_name="c", subcore_axis_name="s"),
        scratch_types=[
            pltpu.VMEM((H, BPW), jnp.int32),
            pltpu.VMEM((BPW, DP), jnp.float32),
            pltpu.VMEM((BPW, DP), jnp.float32),
            pltpu.SemaphoreType.DMA,
            pltpu.SemaphoreType.DMA,
        ],
    )(ids_t, table128)


def _tc_body(acc_ref, ids_ref, w_ref, b_ref, t0_ref, out_ref):
    acc = acc_ref[:, :D]
    ids = ids_ref[...]
    z = jnp.sum((ids == 0).astype(jnp.float32), axis=1, keepdims=True)
    w = w_ref[...]
    t0 = t0_ref[...]
    w0 = lax.dot_general(t0, w, (((1,), (1,)), ((), ())),
                         precision=lax.Precision.HIGHEST,
                         preferred_element_type=jnp.float32)
    y = lax.dot_general(acc, w, (((1,), (1,)), ((), ())),
                        precision=lax.Precision.HIGHEST,
                        preferred_element_type=jnp.float32)
    out_ref[...] = y + b_ref[...] - z * w0


_TC_BLK = 512


def _tc_head(acc, ids, w_out, b_out2, t0):
    return pl.pallas_call(
        _tc_body,
        grid=(B // _TC_BLK,),
        in_specs=[
            pl.BlockSpec((_TC_BLK, DP), lambda i: (i, 0)),
            pl.BlockSpec((_TC_BLK, H), lambda i: (i, 0)),
            pl.BlockSpec((NCLS, D), lambda i: (0, 0)),
            pl.BlockSpec((1, NCLS), lambda i: (0, 0)),
            pl.BlockSpec((1, D), lambda i: (0, 0)),
        ],
        out_specs=pl.BlockSpec((_TC_BLK, NCLS), lambda i: (i, 0)),
        out_shape=jax.ShapeDtypeStruct((B, NCLS), jnp.float32),
    )(acc, ids, w_out, b_out2, t0)


def kernel(words_as_ids, table, W_out, b_out):
    ids = words_as_ids.astype(jnp.int32)
    ids_t = ids.T
    table128 = jnp.pad(table, ((0, 0), (0, DP - D)))
    acc = _sc_pool(ids_t, table128)
    t0 = lax.slice(table, (0, 0), (1, D))
    b2 = b_out.reshape(1, NCLS)
    return _tc_head(acc, ids, W_out, b2, t0)
